# Initial kernel scaffold; baseline (speedup 1.0000x reference)
import jax
import jax.numpy as jnp
from jax.experimental import pallas as pl


def kernel(x):
    raise NotImplementedError("write your pallas kernel here")


# speedup vs baseline: 35.7556x; 35.7556x over previous
import functools

import jax
import jax.numpy as jnp
from jax import lax
from jax.experimental import pallas as pl
from jax.experimental.pallas import tpu as pltpu
from jax.experimental.pallas import tpu_sc as plsc

NB = 2048
LANES = 16
NW = 32
NTOT = 4 * 4096 * 2048
PER_W = NTOT // NW
CHUNK = 32768
NCH = PER_W // CHUNK
VPC = CHUNK // LANES
UNROLL = 8


def _minmax_body(x_ref, min_ref, max_ref):
    i = pl.program_id(0)
    bmin = jnp.min(x_ref[...])
    bmax = jnp.max(x_ref[...])

    @pl.when(i == 0)
    def _():
        min_ref[0, 0] = bmin
        max_ref[0, 0] = bmax

    @pl.when(i != 0)
    def _():
        min_ref[0, 0] = jnp.minimum(min_ref[0, 0], bmin)
        max_ref[0, 0] = jnp.maximum(max_ref[0, 0], bmax)


def _minmax(x2d):
    rows, cols = x2d.shape
    block_rows = 1024
    return pl.pallas_call(
        _minmax_body,
        grid=(rows // block_rows,),
        in_specs=[pl.BlockSpec((block_rows, cols), lambda i: (i, 0))],
        out_specs=[
            pl.BlockSpec(memory_space=pltpu.SMEM),
            pl.BlockSpec(memory_space=pltpu.SMEM),
        ],
        out_shape=[
            jax.ShapeDtypeStruct((1, 1), jnp.float32),
            jax.ShapeDtypeStruct((1, 1), jnp.float32),
        ],
    )(x2d)


def _hist_call(x_flat, params):
    mesh = plsc.VectorSubcoreMesh(core_axis_name="c", subcore_axis_name="s")

    @functools.partial(
        pl.kernel,
        mesh=mesh,
        compiler_params=pltpu.CompilerParams(needs_layout_passes=False),
        out_type=jax.ShapeDtypeStruct((NW * NB,), jnp.float32),
        scratch_types=[
            pltpu.VMEM((CHUNK,), jnp.float32),
            pltpu.VMEM((CHUNK,), jnp.float32),
            pltpu.VMEM((LANES * NB,), jnp.float32),
            pltpu.VMEM((NB,), jnp.float32),
            pltpu.VMEM((2 * LANES,), jnp.float32),
            pltpu.SemaphoreType.DMA,
            pltpu.SemaphoreType.DMA,
        ],
    )
    def hist_kernel(x_hbm, p_hbm, out_hbm, buf0, buf1, hist, outbuf, pbuf,
                    sem0, sem1):
        wid = lax.axis_index("s") * 2 + lax.axis_index("c")
        base = wid * PER_W

        pltpu.sync_copy(p_hbm, pbuf)
        minv = pbuf[pl.ds(0, LANES)]
        wv = pbuf[pl.ds(LANES, LANES)]
        laneoff = lax.iota(jnp.int32, LANES) * NB
        ones = jnp.ones((LANES,), jnp.float32)
        zeros = jnp.zeros((LANES,), jnp.float32)

        def zinit(i, carry):
            hist[pl.ds(i * LANES, LANES)] = zeros
            return carry

        lax.fori_loop(0, NB, zinit, 0)

        def start(c, buf, sem):
            pltpu.async_copy(x_hbm.at[pl.ds(base + c * CHUNK, CHUNK)], buf, sem)

        def wait(buf, sem):
            pltpu.make_async_copy(
                x_hbm.at[pl.ds(base, CHUNK)], buf, sem).wait()

        def process(buf):
            def inner(k, carry):
                for u in range(UNROLL):
                    off = (k * UNROLL + u) * LANES
                    v = buf[pl.ds(off, LANES)]
                    q = (v - minv) / wv
                    idx = q.astype(jnp.int32)
                    idx = jnp.minimum(jnp.maximum(idx, 0), NB - 1)
                    plsc.addupdate_scatter(hist, [idx + laneoff], ones)
                return carry

            lax.fori_loop(0, VPC // UNROLL, inner, 0)

        start(0, buf0, sem0)

        def outer(j, carry):
            start(2 * j + 1, buf1, sem1)
            wait(buf0, sem0)
            process(buf0)

            @pl.when(j < NCH // 2 - 1)
            def _():
                start(2 * j + 2, buf0, sem0)

            wait(buf1, sem1)
            process(buf1)
            return carry

        lax.fori_loop(0, NCH // 2, outer, 0)

        def red(g, carry):
            acc = zeros
            for l in range(LANES):
                acc = acc + hist[pl.ds(l * NB + g * LANES, LANES)]
            outbuf[pl.ds(g * LANES, LANES)] = acc
            return carry

        lax.fori_loop(0, NB // LANES, red, 0)

        pltpu.sync_copy(outbuf, out_hbm.at[pl.ds(wid * NB, NB)])

    return hist_kernel(x_flat, params)


def kernel(x):
    x2d = x.reshape(-1, 2048)
    mn, mx = _minmax(x2d)
    min_val = mn[0, 0]
    max_val = mx[0, 0]
    bin_width = (max_val - min_val) / NB
    safe_width = jnp.where(bin_width == 0, jnp.float32(1.0), bin_width)
    params = jnp.concatenate(
        [jnp.full((LANES,), min_val), jnp.full((LANES,), safe_width)])
    partials = _hist_call(x.reshape(-1), params)
    histogram = partials.reshape(NW, NB).sum(0)
    return (x, histogram, min_val, max_val)


# speedup vs baseline: 114.2464x; 3.1952x over previous
import functools

import jax
import jax.numpy as jnp
from jax import lax
from jax.experimental import pallas as pl
from jax.experimental.pallas import tpu as pltpu
from jax.experimental.pallas import tpu_sc as plsc

NB = 2048
LANES = 16
NW = 32
NTOT = 4 * 4096 * 2048
PER_W = NTOT // NW
CHUNK = 32768
NCH = PER_W // CHUNK
VPC = CHUNK // LANES
UNROLL = 8


def _minmax_body(x_ref, min_ref, max_ref):
    i = pl.program_id(0)
    bmin = jnp.min(x_ref[...])
    bmax = jnp.max(x_ref[...])

    @pl.when(i == 0)
    def _():
        min_ref[0, 0] = bmin
        max_ref[0, 0] = bmax

    @pl.when(i != 0)
    def _():
        min_ref[0, 0] = jnp.minimum(min_ref[0, 0], bmin)
        max_ref[0, 0] = jnp.maximum(max_ref[0, 0], bmax)


def _minmax(x2d):
    rows, cols = x2d.shape
    block_rows = 1024
    return pl.pallas_call(
        _minmax_body,
        grid=(rows // block_rows,),
        in_specs=[pl.BlockSpec((block_rows, cols), lambda i: (i, 0))],
        out_specs=[
            pl.BlockSpec(memory_space=pltpu.SMEM),
            pl.BlockSpec(memory_space=pltpu.SMEM),
        ],
        out_shape=[
            jax.ShapeDtypeStruct((1, 1), jnp.float32),
            jax.ShapeDtypeStruct((1, 1), jnp.float32),
        ],
    )(x2d)


def _hist_call(x_flat, params):
    mesh = plsc.VectorSubcoreMesh(core_axis_name="c", subcore_axis_name="s")

    @functools.partial(
        pl.kernel,
        mesh=mesh,
        compiler_params=pltpu.CompilerParams(needs_layout_passes=False),
        out_type=jax.ShapeDtypeStruct((NW * NB,), jnp.float32),
        scratch_types=[
            pltpu.VMEM((CHUNK,), jnp.float32),
            pltpu.VMEM((CHUNK,), jnp.float32),
            pltpu.VMEM((LANES * NB,), jnp.float32),
            pltpu.VMEM((NB,), jnp.float32),
            pltpu.VMEM((2 * LANES,), jnp.float32),
            pltpu.SemaphoreType.DMA,
            pltpu.SemaphoreType.DMA,
        ],
    )
    def hist_kernel(x_hbm, p_hbm, out_hbm, buf0, buf1, hist, outbuf, pbuf,
                    sem0, sem1):
        wid = lax.axis_index("s") * 2 + lax.axis_index("c")
        base = wid * PER_W

        pltpu.sync_copy(p_hbm, pbuf)
        minv = pbuf[pl.ds(0, LANES)]
        wv = pbuf[pl.ds(LANES, LANES)]
        laneoff = lax.iota(jnp.int32, LANES) * NB
        ones = jnp.ones((LANES,), jnp.float32)
        zeros = jnp.zeros((LANES,), jnp.float32)

        @plsc.parallel_loop(0, LANES * NB // LANES, unroll=8)
        def _zinit(i):
            hist[pl.ds(i * LANES, LANES)] = zeros

        def start(c, buf, sem):
            pltpu.async_copy(x_hbm.at[pl.ds(base + c * CHUNK, CHUNK)], buf, sem)

        def wait(buf, sem):
            pltpu.make_async_copy(
                x_hbm.at[pl.ds(base, CHUNK)], buf, sem).wait()

        def process(buf):
            @plsc.parallel_loop(0, VPC, unroll=UNROLL)
            def _inner(i):
                v = buf[pl.ds(i * LANES, LANES)]
                q = (v - minv) / wv
                idx = jnp.minimum(q.astype(jnp.int32), NB - 1)
                plsc.addupdate_scatter(hist, [idx + laneoff], ones)

        start(0, buf0, sem0)

        def outer(j, carry):
            start(2 * j + 1, buf1, sem1)
            wait(buf0, sem0)
            process(buf0)

            @pl.when(j < NCH // 2 - 1)
            def _():
                start(2 * j + 2, buf0, sem0)

            wait(buf1, sem1)
            process(buf1)
            return carry

        lax.fori_loop(0, NCH // 2, outer, 0)

        @plsc.parallel_loop(0, NB // LANES, unroll=2)
        def _red(g):
            acc = zeros
            for l in range(LANES):
                acc = acc + hist[pl.ds(l * NB + g * LANES, LANES)]
            outbuf[pl.ds(g * LANES, LANES)] = acc

        pltpu.sync_copy(outbuf, out_hbm.at[pl.ds(wid * NB, NB)])

    return hist_kernel(x_flat, params)


def kernel(x):
    x2d = x.reshape(-1, 2048)
    mn, mx = _minmax(x2d)
    min_val = mn[0, 0]
    max_val = mx[0, 0]
    bin_width = (max_val - min_val) / NB
    safe_width = jnp.where(bin_width == 0, jnp.float32(1.0), bin_width)
    params = jnp.concatenate(
        [jnp.full((LANES,), min_val), jnp.full((LANES,), safe_width)])
    partials = _hist_call(x.reshape(-1), params)
    histogram = partials.reshape(NW, NB).sum(0)
    return (x, histogram, min_val, max_val)


# speedup vs baseline: 146.6811x; 1.2839x over previous
import functools

import jax
import jax.numpy as jnp
from jax import lax
from jax.experimental import pallas as pl
from jax.experimental.pallas import tpu as pltpu
from jax.experimental.pallas import tpu_sc as plsc

NB = 2048
LANES = 16
NW = 32
NTOT = 4 * 4096 * 2048
PER_W = NTOT // NW
CHUNK = 32768
NCH = PER_W // CHUNK
VPC = CHUNK // LANES
UNROLL = 8


def _minmax_body(x_ref, min_ref, max_ref):
    i = pl.program_id(0)
    bmin = jnp.min(x_ref[...])
    bmax = jnp.max(x_ref[...])

    @pl.when(i == 0)
    def _():
        min_ref[0, 0] = bmin
        max_ref[0, 0] = bmax

    @pl.when(i != 0)
    def _():
        min_ref[0, 0] = jnp.minimum(min_ref[0, 0], bmin)
        max_ref[0, 0] = jnp.maximum(max_ref[0, 0], bmax)


def _minmax(x2d):
    rows, cols = x2d.shape
    block_rows = 1024
    return pl.pallas_call(
        _minmax_body,
        grid=(rows // block_rows,),
        in_specs=[pl.BlockSpec((block_rows, cols), lambda i: (i, 0))],
        out_specs=[
            pl.BlockSpec(memory_space=pltpu.SMEM),
            pl.BlockSpec(memory_space=pltpu.SMEM),
        ],
        out_shape=[
            jax.ShapeDtypeStruct((1, 1), jnp.float32),
            jax.ShapeDtypeStruct((1, 1), jnp.float32),
        ],
    )(x2d)


def _hist_call(x2d, params):
    mesh = plsc.VectorSubcoreMesh(core_axis_name="c", subcore_axis_name="s")
    rows_per_w = x2d.shape[0] // NW
    chunk_rows = CHUNK // x2d.shape[1]

    @functools.partial(
        pl.kernel,
        mesh=mesh,
        compiler_params=pltpu.CompilerParams(needs_layout_passes=False),
        out_type=jax.ShapeDtypeStruct((NW * NB,), jnp.float32),
        scratch_types=[
            pltpu.VMEM((chunk_rows, NB), jnp.float32),
            pltpu.VMEM((chunk_rows, NB), jnp.float32),
            pltpu.VMEM((LANES * NB,), jnp.float32),
            pltpu.VMEM((NB,), jnp.float32),
            pltpu.VMEM((2 * LANES,), jnp.float32),
            pltpu.SemaphoreType.DMA,
            pltpu.SemaphoreType.DMA,
        ],
    )
    def hist_kernel(x_hbm, p_hbm, out_hbm, buf0, buf1, hist, outbuf, pbuf,
                    sem0, sem1):
        wid = lax.axis_index("s") * 2 + lax.axis_index("c")
        base = wid * rows_per_w

        pltpu.sync_copy(p_hbm, pbuf)
        minv = pbuf[pl.ds(0, LANES)]
        wv = pbuf[pl.ds(LANES, LANES)]
        laneoff = lax.iota(jnp.int32, LANES) * NB
        ones = jnp.ones((LANES,), jnp.float32)
        zeros = jnp.zeros((LANES,), jnp.float32)

        @plsc.parallel_loop(0, LANES * NB // LANES, unroll=8)
        def _zinit(i):
            hist[pl.ds(i * LANES, LANES)] = zeros

        def start(c, buf, sem):
            pltpu.async_copy(
                x_hbm.at[pl.ds(base + c * chunk_rows, chunk_rows), :],
                buf, sem)

        def wait(buf, sem):
            pltpu.make_async_copy(
                x_hbm.at[pl.ds(base, chunk_rows), :], buf, sem).wait()

        def process(buf):
            @plsc.parallel_loop(0, VPC, unroll=UNROLL)
            def _inner(i):
                r = i >> 7
                c = (i & 127) * LANES
                v = buf[r, pl.ds(c, LANES)]
                q = (v - minv) / wv
                idx = jnp.minimum(q.astype(jnp.int32), NB - 1)
                plsc.addupdate_scatter(hist, [idx + laneoff], ones)

        start(0, buf0, sem0)

        def outer(j, carry):
            start(2 * j + 1, buf1, sem1)
            wait(buf0, sem0)
            process(buf0)

            @pl.when(j < NCH // 2 - 1)
            def _():
                start(2 * j + 2, buf0, sem0)

            wait(buf1, sem1)
            process(buf1)
            return carry

        lax.fori_loop(0, NCH // 2, outer, 0)

        @plsc.parallel_loop(0, NB // LANES, unroll=2)
        def _red(g):
            acc = zeros
            for l in range(LANES):
                acc = acc + hist[pl.ds(l * NB + g * LANES, LANES)]
            outbuf[pl.ds(g * LANES, LANES)] = acc

        pltpu.sync_copy(outbuf, out_hbm.at[pl.ds(wid * NB, NB)])

    return hist_kernel(x2d, params)


def kernel(x):
    x2d = x.reshape(-1, 2048)
    mn, mx = _minmax(x2d)
    min_val = mn[0, 0]
    max_val = mx[0, 0]
    bin_width = (max_val - min_val) / NB
    safe_width = jnp.where(bin_width == 0, jnp.float32(1.0), bin_width)
    params = jnp.concatenate(
        [jnp.full((LANES,), min_val), jnp.full((LANES,), safe_width)])
    partials = _hist_call(x2d, params)
    histogram = partials.reshape(NW, NB).sum(0)
    return (x, histogram, min_val, max_val)
